# baseline (speedup 1.0000x reference)
import functools

import jax
import jax.numpy as jnp
from jax import lax
from jax.experimental import pallas as pl
from jax.experimental.pallas import tpu as pltpu
from jax.experimental.pallas import tpu_sc as plsc

B = 64
N = 131072
XMAX = 2048
YMAX = 64

NW = 32
GB = 2
CH = 4096
NCH = N // CH
L = 16


def _sc_body(x_hbm, xc_hbm, sums_hbm, pcnt_hbm,
             idx_v, vals_v, acc_v, mrg_v, cnt_v, sem_i, sem_v):
    cid = lax.axis_index("c")
    sid = lax.axis_index("s")
    wid = sid * 2 + cid

    zf = jnp.zeros((L,), jnp.float32)

    @plsc.parallel_loop(0, XMAX, L, unroll=8)
    def _(i):
        cnt_v[pl.ds(i, L)] = zf

    @plsc.parallel_loop(0, GB * XMAX * L, L, unroll=8)
    def _(i):
        acc_v[pl.ds(i, L)] = zf

    def start(c, buf):
        base = c * CH
        pltpu.make_async_copy(
            xc_hbm.at[pl.ds(base, CH)], idx_v.at[buf], sem_i.at[buf]).start()
        pltpu.make_async_copy(
            x_hbm.at[pl.ds(wid * GB, GB), 0, 0, pl.ds(base, CH)],
            vals_v.at[buf], sem_v.at[buf]).start()

    def wait(c, buf):
        base = c * CH
        pltpu.make_async_copy(
            xc_hbm.at[pl.ds(base, CH)], idx_v.at[buf], sem_i.at[buf]).wait()
        pltpu.make_async_copy(
            x_hbm.at[pl.ds(wid * GB, GB), 0, 0, pl.ds(base, CH)],
            vals_v.at[buf], sem_v.at[buf]).wait()

    ones = jnp.full((L,), 1.0, jnp.float32)
    lanes = lax.iota(jnp.int32, L)
    start(0, 0)
    for c in range(NCH):
        buf = c & 1
        if c + 1 < NCH:
            start(c + 1, (c + 1) & 1)
        wait(c, buf)

        @plsc.parallel_loop(0, CH, L, unroll=4)
        def _(i):
            iv = idx_v[buf, pl.ds(i, L)]
            jv = (iv << 4) + lanes
            for b in range(GB):
                v = vals_v[buf, b, pl.ds(i, L)]
                plsc.addupdate_scatter(
                    acc_v, [jv + jnp.int32(b * XMAX * L)], v)

        @pl.when(wid == c)
        def _():
            @plsc.parallel_loop(0, CH, L, unroll=4)
            def _(i):
                iv = idx_v[buf, pl.ds(i, L)]
                plsc.addupdate_scatter(cnt_v, [iv], ones)

    m15 = lax.iota(jnp.int32, L) == jnp.int32(L - 1)

    @plsc.parallel_loop(0, GB * XMAX, 1, unroll=8)
    def _(j):
        cs = plsc.cumsum(acc_v[pl.ds(j * L, L)])
        plsc.store_compressed(mrg_v.at[pl.ds(j, L)], cs, mask=m15)

    for b in range(GB):
        pltpu.sync_copy(mrg_v.at[pl.ds(b * XMAX, XMAX)],
                        sums_hbm.at[wid * GB + b])
    pltpu.sync_copy(cnt_v, pcnt_hbm.at[wid])


_sc_segsum = functools.partial(
    pl.kernel,
    out_type=(
        jax.ShapeDtypeStruct((B, XMAX), jnp.float32),
        jax.ShapeDtypeStruct((NW, XMAX), jnp.float32),
    ),
    mesh=plsc.VectorSubcoreMesh(core_axis_name="c", subcore_axis_name="s"),
    compiler_params=pltpu.CompilerParams(needs_layout_passes=False),
    scratch_types=[
        pltpu.VMEM((2, CH), jnp.int32),
        pltpu.VMEM((2, GB, CH), jnp.float32),
        pltpu.VMEM((GB * XMAX * L,), jnp.float32),
        pltpu.VMEM((GB * XMAX + L,), jnp.float32),
        pltpu.VMEM((XMAX,), jnp.float32),
        pltpu.SemaphoreType.DMA((2,)),
        pltpu.SemaphoreType.DMA((2,)),
    ],
)(_sc_body)


BT = 8


def _tc_body(sums_ref, pcnt_ref, y_ref, out_ref):
    cnt = jnp.maximum(
        jnp.sum(pcnt_ref[...], axis=0, keepdims=True), 1.0)
    inv = 1.0 / cnt
    yv = y_ref[0:1, :]
    yi = lax.broadcasted_iota(jnp.int32, (YMAX, XMAX), 0)
    m = yi == yv
    for b in range(BT):
        srow = sums_ref[b:b + 1, :]
        out_ref[b] = jnp.where(m, srow * inv, 0.0)


def _tc_expand(sums, pcnt, y2):
    return pl.pallas_call(
        _tc_body,
        grid=(B // BT,),
        in_specs=[
            pl.BlockSpec((BT, XMAX), lambda b: (b, 0)),
            pl.BlockSpec((NW, XMAX), lambda b: (0, 0)),
            pl.BlockSpec((8, XMAX), lambda b: (0, 0)),
        ],
        out_specs=pl.BlockSpec((BT, YMAX, XMAX), lambda b: (b, 0, 0)),
        out_shape=jax.ShapeDtypeStruct((B, YMAX, XMAX), jnp.float32),
    )(sums, pcnt, y2)


def kernel(x, x_coord, y_coord):
    sums, pcnt = _sc_segsum(x, x_coord)
    y2 = jnp.broadcast_to(y_coord.reshape(1, XMAX), (8, XMAX))
    return _tc_expand(sums, pcnt, y2)


# speedup vs baseline: 1.1387x; 1.1387x over previous
import functools

import jax
import jax.numpy as jnp
from jax import lax
from jax.experimental import pallas as pl
from jax.experimental.pallas import tpu as pltpu
from jax.experimental.pallas import tpu_sc as plsc

B = 64
N = 131072
XMAX = 2048
YMAX = 64

NW = 32
GB = 8
NG = B // GB
NS = NW // NG
SLICE = N // NS
CH = 4096
NCH = SLICE // CH
L = 16


def _sc_body(x_hbm, xc_hbm, psums_hbm, pcnt_hbm,
             idx_v, vals_v, acc_v, cnt_v, sem_i, sem_v):
    cid = lax.axis_index("c")
    sid = lax.axis_index("s")
    wid = sid * 2 + cid
    g = wid % NG
    s = wid // NG

    zf = jnp.zeros((L,), jnp.float32)

    @plsc.parallel_loop(0, XMAX, L, unroll=8)
    def _(i):
        cnt_v[pl.ds(i, L)] = zf

    @plsc.parallel_loop(0, GB * XMAX, L, unroll=8)
    def _(i):
        acc_v[pl.ds(i, L)] = zf

    def start(c, buf):
        base = s * SLICE + c * CH
        pltpu.make_async_copy(
            xc_hbm.at[pl.ds(base, CH)], idx_v.at[buf], sem_i.at[buf]).start()
        pltpu.make_async_copy(
            x_hbm.at[pl.ds(g * GB, GB), 0, 0, pl.ds(base, CH)],
            vals_v.at[buf], sem_v.at[buf]).start()

    def wait(c, buf):
        base = s * SLICE + c * CH
        pltpu.make_async_copy(
            xc_hbm.at[pl.ds(base, CH)], idx_v.at[buf], sem_i.at[buf]).wait()
        pltpu.make_async_copy(
            x_hbm.at[pl.ds(g * GB, GB), 0, 0, pl.ds(base, CH)],
            vals_v.at[buf], sem_v.at[buf]).wait()

    ones = jnp.full((L,), 1.0, jnp.float32)
    start(0, 0)
    for c in range(NCH):
        buf = c & 1
        if c + 1 < NCH:
            start(c + 1, (c + 1) & 1)
        wait(c, buf)

        @plsc.parallel_loop(0, CH, L, unroll=4)
        def _(i):
            iv = idx_v[buf, pl.ds(i, L)]
            for b in range(GB):
                v = vals_v[buf, b, pl.ds(i, L)]
                plsc.addupdate_scatter(acc_v, [iv + jnp.int32(b * XMAX)], v)

        @pl.when(g == c)
        def _():
            @plsc.parallel_loop(0, CH, L, unroll=4)
            def _(i):
                iv = idx_v[buf, pl.ds(i, L)]
                plsc.addupdate_scatter(cnt_v, [iv], ones)

    for b in range(GB):
        pltpu.sync_copy(acc_v.at[pl.ds(b * XMAX, XMAX)],
                        psums_hbm.at[g * GB + b, s])
    pltpu.sync_copy(cnt_v, pcnt_hbm.at[wid])


_sc_segsum = functools.partial(
    pl.kernel,
    out_type=(
        jax.ShapeDtypeStruct((B, NS, XMAX), jnp.float32),
        jax.ShapeDtypeStruct((NW, XMAX), jnp.float32),
    ),
    mesh=plsc.VectorSubcoreMesh(core_axis_name="c", subcore_axis_name="s"),
    compiler_params=pltpu.CompilerParams(needs_layout_passes=False, use_tc_tiling_on_sc=True),
    scratch_types=[
        pltpu.VMEM((2, CH), jnp.int32),
        pltpu.VMEM((2, GB, CH), jnp.float32),
        pltpu.VMEM((GB * XMAX,), jnp.float32),
        pltpu.VMEM((XMAX,), jnp.float32),
        pltpu.SemaphoreType.DMA((2,)),
        pltpu.SemaphoreType.DMA((2,)),
    ],
)(_sc_body)


BT = 16


def _tc_body(psums_ref, pcnt_ref, y_ref, out_ref):
    cnt = jnp.maximum(
        jnp.sum(pcnt_ref[...], axis=0, keepdims=True), 1.0)
    inv = 1.0 / cnt
    yv = y_ref[0:1, :]
    yi = lax.broadcasted_iota(jnp.int32, (YMAX, XMAX), 0)
    m = yi == yv
    for b in range(BT):
        srow = jnp.sum(psums_ref[b], axis=0, keepdims=True)
        out_ref[b] = jnp.where(m, srow * inv, 0.0)


def _tc_expand(psums, pcnt, y2):
    return pl.pallas_call(
        _tc_body,
        grid=(B // BT,),
        in_specs=[
            pl.BlockSpec((BT, NS, XMAX), lambda b: (b, 0, 0)),
            pl.BlockSpec((NW, XMAX), lambda b: (0, 0)),
            pl.BlockSpec((8, XMAX), lambda b: (0, 0)),
        ],
        out_specs=pl.BlockSpec((BT, YMAX, XMAX), lambda b: (b, 0, 0)),
        out_shape=jax.ShapeDtypeStruct((B, YMAX, XMAX), jnp.float32),
    )(psums, pcnt, y2)


def kernel(x, x_coord, y_coord):
    psums, pcnt = _sc_segsum(x, x_coord)
    y2 = jnp.broadcast_to(y_coord.reshape(1, XMAX), (8, XMAX))
    return _tc_expand(psums, pcnt, y2)


# speedup vs baseline: 1.1449x; 1.0054x over previous
import functools

import jax
import jax.numpy as jnp
from jax import lax
from jax.experimental import pallas as pl
from jax.experimental.pallas import tpu as pltpu
from jax.experimental.pallas import tpu_sc as plsc

B = 64
N = 131072
XMAX = 2048
YMAX = 64

NW = 32
GB = 8
NG = B // GB
NS = NW // NG
SLICE = N // NS
CH = 4096
NCH = SLICE // CH
L = 16


def _sc_body(x_hbm, xc_hbm, psums_hbm, pcnt_hbm,
             idx_v, vals_v, acc_v, cnt_v, sem_i, sem_v, sem_o):
    cid = lax.axis_index("c")
    sid = lax.axis_index("s")
    wid = sid * 2 + cid
    g = wid % NG
    s = wid // NG

    zf = jnp.zeros((L,), jnp.float32)

    @plsc.parallel_loop(0, XMAX, L, unroll=8)
    def _(i):
        cnt_v[pl.ds(i, L)] = zf

    @plsc.parallel_loop(0, GB * XMAX, L, unroll=8)
    def _(i):
        acc_v[pl.ds(i, L)] = zf

    def start(c, buf):
        base = s * SLICE + c * CH
        pltpu.make_async_copy(
            xc_hbm.at[pl.ds(base, CH)], idx_v.at[buf], sem_i.at[buf]).start()
        pltpu.make_async_copy(
            x_hbm.at[pl.ds(g * GB, GB), 0, 0, pl.ds(base, CH)],
            vals_v.at[buf], sem_v.at[buf]).start()

    def wait(c, buf):
        base = s * SLICE + c * CH
        pltpu.make_async_copy(
            xc_hbm.at[pl.ds(base, CH)], idx_v.at[buf], sem_i.at[buf]).wait()
        pltpu.make_async_copy(
            x_hbm.at[pl.ds(g * GB, GB), 0, 0, pl.ds(base, CH)],
            vals_v.at[buf], sem_v.at[buf]).wait()

    ones = jnp.full((L,), 1.0, jnp.float32)
    start(0, 0)
    for c in range(NCH):
        buf = c & 1
        if c + 1 < NCH:
            start(c + 1, (c + 1) & 1)
        wait(c, buf)

        @plsc.parallel_loop(0, CH, L, unroll=4)
        def _(i):
            iv = idx_v[buf, pl.ds(i, L)]
            for b in range(GB):
                v = vals_v[buf, b, pl.ds(i, L)]
                plsc.addupdate_scatter(acc_v, [iv + jnp.int32(b * XMAX)], v)

        @pl.when(g == c)
        def _():
            @plsc.parallel_loop(0, CH, L, unroll=4)
            def _(i):
                iv = idx_v[buf, pl.ds(i, L)]
                plsc.addupdate_scatter(cnt_v, [iv], ones)

    outs = [pltpu.make_async_copy(acc_v.at[pl.ds(b * XMAX, XMAX)],
                                  psums_hbm.at[g * GB + b, s], sem_o)
            for b in range(GB)]
    outs.append(pltpu.make_async_copy(cnt_v, pcnt_hbm.at[wid], sem_o))
    for o in outs:
        o.start()
    for o in outs:
        o.wait()


_sc_segsum = functools.partial(
    pl.kernel,
    out_type=(
        jax.ShapeDtypeStruct((B, NS, XMAX), jnp.float32),
        jax.ShapeDtypeStruct((NW, XMAX), jnp.float32),
    ),
    mesh=plsc.VectorSubcoreMesh(core_axis_name="c", subcore_axis_name="s"),
    compiler_params=pltpu.CompilerParams(needs_layout_passes=False, use_tc_tiling_on_sc=True),
    scratch_types=[
        pltpu.VMEM((2, CH), jnp.int32),
        pltpu.VMEM((2, GB, CH), jnp.float32),
        pltpu.VMEM((GB * XMAX,), jnp.float32),
        pltpu.VMEM((XMAX,), jnp.float32),
        pltpu.SemaphoreType.DMA((2,)),
        pltpu.SemaphoreType.DMA((2,)),
        pltpu.SemaphoreType.DMA,
    ],
)(_sc_body)


BT = 16


def _tc_body(psums_ref, pcnt_ref, y_ref, out_ref):
    cnt = jnp.maximum(
        jnp.sum(pcnt_ref[...], axis=0, keepdims=True), 1.0)
    inv = 1.0 / cnt
    yv = y_ref[0:1, :]
    yi = lax.broadcasted_iota(jnp.int32, (YMAX, XMAX), 0)
    m = yi == yv
    for b in range(BT):
        srow = jnp.sum(psums_ref[b], axis=0, keepdims=True)
        out_ref[b] = jnp.where(m, srow * inv, 0.0)


def _tc_expand(psums, pcnt, y2):
    return pl.pallas_call(
        _tc_body,
        grid=(B // BT,),
        in_specs=[
            pl.BlockSpec((BT, NS, XMAX), lambda b: (b, 0, 0)),
            pl.BlockSpec((NW, XMAX), lambda b: (0, 0)),
            pl.BlockSpec((8, XMAX), lambda b: (0, 0)),
        ],
        out_specs=pl.BlockSpec((BT, YMAX, XMAX), lambda b: (b, 0, 0)),
        out_shape=jax.ShapeDtypeStruct((B, YMAX, XMAX), jnp.float32),
    )(psums, pcnt, y2)


def kernel(x, x_coord, y_coord):
    psums, pcnt = _sc_segsum(x, x_coord)
    y2 = jnp.broadcast_to(y_coord.reshape(1, XMAX), (8, XMAX))
    return _tc_expand(psums, pcnt, y2)
